# initial kernel scaffold (unmeasured)
import jax
import jax.numpy as jnp
from jax import lax
from jax.experimental import pallas as pl
from jax.experimental.pallas import tpu as pltpu

N_DEV = 8
M_PER = 512
N_OUT = 2048


def _gelu(y):
    c = 0.7978845608028654
    return 0.5 * y * (1.0 + jnp.tanh(c * (y + 0.044715 * y * y * y)))


def kernel(x, w_mat):
    def body(x_ref, w_ref, out_ref, comm_ref, send_sems, recv_sems):
        my = lax.axis_index("i")
        left = lax.rem(my + N_DEV - 1, N_DEV)
        right = lax.rem(my + 1, N_DEV)

        barrier_sem = pltpu.get_barrier_semaphore()
        for nbr in (left, right):
            pl.semaphore_signal(
                barrier_sem, inc=1,
                device_id=(nbr,), device_id_type=pl.DeviceIdType.MESH,
            )
        pl.semaphore_wait(barrier_sem, 2)

        def partial_chunk(c):
            xs = x_ref[pl.ds(c * M_PER, M_PER), :]
            return jnp.dot(xs, w_ref[:, :], preferred_element_type=jnp.float32)

        comm_ref[0, :, :] = partial_chunk(left)

        for s in range(N_DEV - 1):
            rdma = pltpu.make_async_remote_copy(
                src_ref=comm_ref.at[s],
                dst_ref=comm_ref.at[s + 1],
                send_sem=send_sems.at[s],
                recv_sem=recv_sems.at[s],
                device_id=(right,),
                device_id_type=pl.DeviceIdType.MESH,
            )
            rdma.start()
            rdma.wait()

            c = lax.rem(my + 2 * N_DEV - s - 2, N_DEV)
            acc = comm_ref[s + 1, :, :] + partial_chunk(c)
            if s < N_DEV - 2:
                comm_ref[s + 1, :, :] = acc
            else:
                out_ref[:, :] = _gelu(acc)

    return pl.pallas_call(
        body,
        out_shape=jax.ShapeDtypeStruct((M_PER, N_OUT), jnp.float32),
        in_specs=[
            pl.BlockSpec(memory_space=pltpu.VMEM),
            pl.BlockSpec(memory_space=pltpu.VMEM),
        ],
        out_specs=pl.BlockSpec(memory_space=pltpu.VMEM),
        scratch_shapes=[
            pltpu.VMEM((N_DEV, M_PER, N_OUT), jnp.float32),
            pltpu.SemaphoreType.DMA((N_DEV - 1,)),
            pltpu.SemaphoreType.DMA((N_DEV - 1,)),
        ],
        compiler_params=pltpu.CompilerParams(collective_id=0),
    )(x, w_mat)


# baseline (device time: 361622 ns/iter reference)
import jax
import jax.numpy as jnp
from jax import lax
from jax.experimental import pallas as pl
from jax.experimental.pallas import tpu as pltpu

N_DEV = 8
M_PER = 512
N_OUT = 2048


def _gelu(y):
    c = 0.7978845608028654
    return 0.5 * y * (1.0 + jnp.tanh(c * (y + 0.044715 * y * y * y)))


def kernel(x, w_mat):
    x = x.astype(jnp.bfloat16)
    w_mat = w_mat.astype(jnp.bfloat16)

    def body(x_ref, w_ref, out_ref, comm_ref, send_sems, recv_sems):
        my = lax.axis_index("i")
        left = lax.rem(my + N_DEV - 1, N_DEV)
        right = lax.rem(my + 1, N_DEV)

        barrier_sem = pltpu.get_barrier_semaphore()
        for nbr in (left, right):
            pl.semaphore_signal(
                barrier_sem, inc=1,
                device_id=(nbr,), device_id_type=pl.DeviceIdType.MESH,
            )
        pl.semaphore_wait(barrier_sem, 2)

        def partial_chunk(c):
            xs = x_ref[pl.ds(c * M_PER, M_PER), :]
            return jnp.dot(xs, w_ref[:, :], preferred_element_type=jnp.float32)

        comm_ref[0, :, :] = partial_chunk(left)

        for s in range(N_DEV - 1):
            rdma = pltpu.make_async_remote_copy(
                src_ref=comm_ref.at[s],
                dst_ref=comm_ref.at[s + 1],
                send_sem=send_sems.at[s],
                recv_sem=recv_sems.at[s],
                device_id=(right,),
                device_id_type=pl.DeviceIdType.MESH,
            )
            rdma.start()
            rdma.wait()

            c = lax.rem(my + 2 * N_DEV - s - 2, N_DEV)
            acc = comm_ref[s + 1, :, :] + partial_chunk(c)
            if s < N_DEV - 2:
                comm_ref[s + 1, :, :] = acc
            else:
                out_ref[:, :] = _gelu(acc)

    return pl.pallas_call(
        body,
        out_shape=jax.ShapeDtypeStruct((M_PER, N_OUT), jnp.float32),
        in_specs=[
            pl.BlockSpec(memory_space=pltpu.VMEM),
            pl.BlockSpec(memory_space=pltpu.VMEM),
        ],
        out_specs=pl.BlockSpec(memory_space=pltpu.VMEM),
        scratch_shapes=[
            pltpu.VMEM((N_DEV, M_PER, N_OUT), jnp.float32),
            pltpu.SemaphoreType.DMA((N_DEV - 1,)),
            pltpu.SemaphoreType.DMA((N_DEV - 1,)),
        ],
        compiler_params=pltpu.CompilerParams(
            collective_id=0, vmem_limit_bytes=100 * 1024 * 1024
        ),
    )(x, w_mat)


# device time: 123802 ns/iter; 2.9210x vs baseline; 2.9210x over previous
import jax
import jax.numpy as jnp
from jax import lax
from jax.experimental import pallas as pl
from jax.experimental.pallas import tpu as pltpu

N_DEV = 8
M_PER = 512
N_OUT = 2048
HALF = N_OUT // 2


def _gelu(y):
    c = 0.7978845608028654
    return 0.5 * y * (1.0 + jnp.tanh(c * (y + 0.044715 * y * y * y)))


def kernel(x, w_mat):
    x = x.astype(jnp.bfloat16)
    w_mat = w_mat.astype(jnp.bfloat16)

    def body(x_ref, w_ref, out_ref, fwd_ref, bwd_ref,
             fsend, frecv, bsend, brecv):
        my = lax.axis_index("i")
        left = lax.rem(my + N_DEV - 1, N_DEV)
        right = lax.rem(my + 1, N_DEV)

        barrier_sem = pltpu.get_barrier_semaphore()
        for nbr in (left, right):
            pl.semaphore_signal(
                barrier_sem, inc=1,
                device_id=(nbr,), device_id_type=pl.DeviceIdType.MESH,
            )
        pl.semaphore_wait(barrier_sem, 2)

        def partial_half(c, lo):
            xs = x_ref[pl.ds(c * M_PER, M_PER), :]
            return jnp.dot(
                xs, w_ref[:, lo:lo + HALF], preferred_element_type=jnp.float32
            )

        fwd_ref[0, :, :] = partial_half(left, 0).astype(jnp.bfloat16)
        bwd_ref[0, :, :] = partial_half(right, HALF).astype(jnp.bfloat16)

        for s in range(N_DEV - 1):
            f_rdma = pltpu.make_async_remote_copy(
                src_ref=fwd_ref.at[s],
                dst_ref=fwd_ref.at[s + 1],
                send_sem=fsend.at[s],
                recv_sem=frecv.at[s],
                device_id=(right,),
                device_id_type=pl.DeviceIdType.MESH,
            )
            b_rdma = pltpu.make_async_remote_copy(
                src_ref=bwd_ref.at[s],
                dst_ref=bwd_ref.at[s + 1],
                send_sem=bsend.at[s],
                recv_sem=brecv.at[s],
                device_id=(left,),
                device_id_type=pl.DeviceIdType.MESH,
            )
            f_rdma.start()
            b_rdma.start()

            cf = lax.rem(my + 2 * N_DEV - s - 2, N_DEV)
            cb = lax.rem(my + s + 2, N_DEV)
            pf = partial_half(cf, 0)
            pb = partial_half(cb, HALF)

            f_rdma.wait()
            b_rdma.wait()
            facc = fwd_ref[s + 1, :, :].astype(jnp.float32) + pf
            bacc = bwd_ref[s + 1, :, :].astype(jnp.float32) + pb
            if s < N_DEV - 2:
                fwd_ref[s + 1, :, :] = facc.astype(jnp.bfloat16)
                bwd_ref[s + 1, :, :] = bacc.astype(jnp.bfloat16)
            else:
                out_ref[:, :HALF] = _gelu(facc)
                out_ref[:, HALF:] = _gelu(bacc)

    return pl.pallas_call(
        body,
        out_shape=jax.ShapeDtypeStruct((M_PER, N_OUT), jnp.float32),
        in_specs=[
            pl.BlockSpec(memory_space=pltpu.VMEM),
            pl.BlockSpec(memory_space=pltpu.VMEM),
        ],
        out_specs=pl.BlockSpec(memory_space=pltpu.VMEM),
        scratch_shapes=[
            pltpu.VMEM((N_DEV, M_PER, HALF), jnp.bfloat16),
            pltpu.VMEM((N_DEV, M_PER, HALF), jnp.bfloat16),
            pltpu.SemaphoreType.DMA((N_DEV - 1,)),
            pltpu.SemaphoreType.DMA((N_DEV - 1,)),
            pltpu.SemaphoreType.DMA((N_DEV - 1,)),
            pltpu.SemaphoreType.DMA((N_DEV - 1,)),
        ],
        compiler_params=pltpu.CompilerParams(
            collective_id=0, vmem_limit_bytes=100 * 1024 * 1024
        ),
    )(x, w_mat)


# device time: 105301 ns/iter; 3.4342x vs baseline; 1.1757x over previous
import jax
import jax.numpy as jnp
from jax import lax
from jax.experimental import pallas as pl
from jax.experimental.pallas import tpu as pltpu

N_DEV = 8
M_PER = 512
N_OUT = 2048
HALF = N_OUT // 2
SEG = M_PER // 2


def _gelu(y):
    c = 0.7978845608028654
    return 0.5 * y * (1.0 + jnp.tanh(c * (y + 0.044715 * y * y * y)))


def kernel(x, w_mat):
    x = x.astype(jnp.bfloat16)
    w_mat = w_mat.astype(jnp.bfloat16)

    def body(x_ref, w_ref, out_ref, fwd_ref, bwd_ref,
             fsend, frecv, bsend, brecv):
        my = lax.axis_index("i")
        left = lax.rem(my + N_DEV - 1, N_DEV)
        right = lax.rem(my + 1, N_DEV)

        barrier_sem = pltpu.get_barrier_semaphore()
        for nbr in (left, right):
            pl.semaphore_signal(
                barrier_sem, inc=1,
                device_id=(nbr,), device_id_type=pl.DeviceIdType.MESH,
            )
        pl.semaphore_wait(barrier_sem, 2)

        def partial_half(c, lo):
            xs = x_ref[pl.ds(c * M_PER, M_PER), :]
            return jnp.dot(
                xs, w_ref[:, lo:lo + HALF], preferred_element_type=jnp.float32
            )

        def mk(ring_ref, send_sems, recv_sems, nbr, s, g):
            return pltpu.make_async_remote_copy(
                src_ref=ring_ref.at[s, g * SEG:(g + 1) * SEG, :],
                dst_ref=ring_ref.at[s + 1, g * SEG:(g + 1) * SEG, :],
                send_sem=send_sems.at[s, g],
                recv_sem=recv_sems.at[s, g],
                device_id=(nbr,),
                device_id_type=pl.DeviceIdType.MESH,
            )

        def mk_f(s, g):
            return mk(fwd_ref, fsend, frecv, right, s, g)

        def mk_b(s, g):
            return mk(bwd_ref, bsend, brecv, left, s, g)

        fwd_ref[0, :, :] = partial_half(left, 0).astype(jnp.bfloat16)
        mk_f(0, 0).start()
        mk_f(0, 1).start()
        bwd_ref[0, :, :] = partial_half(right, HALF).astype(jnp.bfloat16)
        mk_b(0, 0).start()
        mk_b(0, 1).start()

        for s in range(N_DEV - 1):
            cf = lax.rem(my + 2 * N_DEV - s - 2, N_DEV)
            cb = lax.rem(my + s + 2, N_DEV)
            pf = partial_half(cf, 0)
            pb = partial_half(cb, HALF)

            for g in range(2):
                rows = slice(g * SEG, (g + 1) * SEG)
                mk_f(s, g).wait()
                facc = fwd_ref[s + 1, rows, :].astype(jnp.float32) + pf[rows, :]
                if s < N_DEV - 2:
                    fwd_ref[s + 1, rows, :] = facc.astype(jnp.bfloat16)
                    mk_f(s + 1, g).start()
                else:
                    out_ref[rows, :HALF] = _gelu(facc)
                mk_b(s, g).wait()
                bacc = bwd_ref[s + 1, rows, :].astype(jnp.float32) + pb[rows, :]
                if s < N_DEV - 2:
                    bwd_ref[s + 1, rows, :] = bacc.astype(jnp.bfloat16)
                    mk_b(s + 1, g).start()
                else:
                    out_ref[rows, HALF:] = _gelu(bacc)

    return pl.pallas_call(
        body,
        out_shape=jax.ShapeDtypeStruct((M_PER, N_OUT), jnp.float32),
        in_specs=[
            pl.BlockSpec(memory_space=pltpu.VMEM),
            pl.BlockSpec(memory_space=pltpu.VMEM),
        ],
        out_specs=pl.BlockSpec(memory_space=pltpu.VMEM),
        scratch_shapes=[
            pltpu.VMEM((N_DEV, M_PER, HALF), jnp.bfloat16),
            pltpu.VMEM((N_DEV, M_PER, HALF), jnp.bfloat16),
            pltpu.SemaphoreType.DMA((N_DEV - 1, 2)),
            pltpu.SemaphoreType.DMA((N_DEV - 1, 2)),
            pltpu.SemaphoreType.DMA((N_DEV - 1, 2)),
            pltpu.SemaphoreType.DMA((N_DEV - 1, 2)),
        ],
        compiler_params=pltpu.CompilerParams(
            collective_id=0, vmem_limit_bytes=100 * 1024 * 1024
        ),
    )(x, w_mat)
